# baseline (speedup 1.0000x reference)
import functools

import jax
import jax.numpy as jnp
from jax import lax
from jax.experimental import pallas as pl
from jax.experimental.pallas import tpu as pltpu
from jax.experimental.pallas import tpu_sc as plsc

N = 100000
D = 128
G = 512
B = 5000
NB = N // B
OFF_PAD = 544
HI = 4
LO = 128

NC = 2
NS = 16
NW = NC * NS
SPW = G // NW
C = 384
GRPS = C // 16
NREG = 17


def _tc_body(x_ref, b_ref, wg_ref, bg_ref, wgate_ref, bgate_ref,
             logit_ref, off_ref, acc_c):
    i = pl.program_id(0)

    @pl.when(i == 0)
    def _init():
        acc_c[...] = jnp.zeros_like(acc_c)

    dvec = jnp.concatenate(
        [jnp.full((1, 1), -1.0, jnp.float32),
         jnp.full((1, 1), 1.0, jnp.float32)], axis=0)
    wd = jax.lax.dot(wgate_ref[...], dvec,
                     preferred_element_type=jnp.float32)
    bd = jax.lax.dot(bgate_ref[...], dvec,
                     preferred_element_type=jnp.float32)
    vt = jax.lax.dot_general(
        wd, wg_ref[...], (((0,), (1,)), ((), ())),
        preferred_element_type=jnp.float32)
    c = jax.lax.dot_general(
        bg_ref[...], vt, (((1,), (1,)), ((), ())),
        preferred_element_type=jnp.float32)
    logit_t = (jax.lax.dot_general(
                   vt, x_ref[...], (((1,), (1,)), ((), ())),
                   preferred_element_type=jnp.float32)
               + (c[0, 0] + bd[0, 0]))
    logit_ref[0] = logit_t

    ids = b_ref[0]
    hi_t = ids >> 7
    lo_t = ids & 127
    ihi = jax.lax.broadcasted_iota(jnp.int32, (HI, B), 0)
    ilo = jax.lax.broadcasted_iota(jnp.int32, (LO, B), 0)
    ohhi = (hi_t == ihi).astype(jnp.float32)
    ohlo = (lo_t == ilo).astype(jnp.float32)
    acc_c[...] += jax.lax.dot_general(
        ohhi, ohlo, (((1,), (1,)), ((), ())),
        preferred_element_type=jnp.float32)

    @pl.when(i == NB - 1)
    def _finalize():
        shi = jax.lax.broadcasted_iota(jnp.int32, (G, HI), 0) >> 7
        ih = jax.lax.broadcasted_iota(jnp.int32, (G, HI), 1)
        p = (shi == ih).astype(jnp.float32)
        slo = jax.lax.broadcasted_iota(jnp.int32, (G, LO), 0) & 127
        il = jax.lax.broadcasted_iota(jnp.int32, (G, LO), 1)
        rsel = (slo == il).astype(jnp.float32)
        t = jax.lax.dot(p, acc_c[...],
                        preferred_element_type=jnp.float32)
        count = jnp.sum(t * rsel, axis=1, keepdims=True)
        it = jax.lax.broadcasted_iota(jnp.int32, (G, OFF_PAD), 0)
        ic = jax.lax.broadcasted_iota(jnp.int32, (G, OFF_PAD), 1)
        m = (it < ic).astype(jnp.float32)
        off = jax.lax.dot_general(
            count, m, (((0,), (0,)), ((), ())),
            preferred_element_type=jnp.float32)
        off_ref[...] = off.astype(jnp.int32)


def _tc_call(x_in, batch_3d, W_gnn, b_gnn2, W_gate, b_gate2):
    out_shapes = (
        jax.ShapeDtypeStruct((NB, 1, B), jnp.float32),
        jax.ShapeDtypeStruct((1, OFF_PAD), jnp.int32),
    )
    in_specs = [
        pl.BlockSpec((B, D), lambda i: (i, 0)),
        pl.BlockSpec((1, 1, B), lambda i: (i, 0, 0)),
        pl.BlockSpec((D, D), lambda i: (0, 0)),
        pl.BlockSpec((1, D), lambda i: (0, 0)),
        pl.BlockSpec((D, 2), lambda i: (0, 0)),
        pl.BlockSpec((1, 2), lambda i: (0, 0)),
    ]
    out_specs = (
        pl.BlockSpec((1, 1, B), lambda i: (i, 0, 0)),
        pl.BlockSpec((1, OFF_PAD), lambda i: (0, 0)),
    )
    return pl.pallas_call(
        _tc_body,
        grid=(NB,),
        in_specs=in_specs,
        out_specs=out_specs,
        out_shape=out_shapes,
        scratch_shapes=[pltpu.VMEM((HI, LO), jnp.float32)],
    )(x_in, batch_3d, W_gnn, b_gnn2, W_gate, b_gate2)


_IOTA = lambda: lax.iota(jnp.int32, 16)


def _ext_i(vec, j):
    return jnp.sum(jnp.where(_IOTA() == j, vec, 0))


def _ext_f(vec, j):
    return jnp.sum(jnp.where(_IOTA() == j, vec, 0.0))


def _bcast_lane(vec, j):
    idx = jnp.full((16,), j, jnp.int32)
    return jnp.take_along_axis(vec, idx, axis=0)


def _zero_regs():
    return tuple(jnp.zeros((16,), jnp.float32) for _ in range(NREG))


def _sigmoid16(x):
    return jnp.ones((16,), jnp.float32) / (1.0 + jnp.exp(-x))


def _sc_body(h_hbm, logit_hbm, batch_hbm, gum_hbm, off_hbm,
             hout_hbm, cout_hbm, r_hbm, env_hbm, gate_hbm,
             h_va, h_vb, b_va, b_vb, l_va, l_vb, gm_va, gm_vb, go_va, go_vb,
             off_vmem, acc, hbuf, cbuf, rbuf,
             sem_a, sem_b):
    cid = lax.axis_index("c")
    sid = lax.axis_index("s")
    w = sid * NC + cid
    seg_lo = w * SPW

    pltpu.sync_copy(off_hbm.at[pl.ds(seg_lo, 32)], off_vmem)
    ovec0 = off_vmem[pl.ds(0, 16)]
    ovec1 = off_vmem[pl.ds(16, 16)]
    starts = [_ext_i(ovec0, j) for j in range(16)] + [_ext_i(ovec1, 0)]
    row_lo = starts[0]
    row_hi = starts[SPW]
    a0 = (row_lo // 16) * 16
    z0 = ((row_hi + 15) // 16) * 16
    nch = (z0 - a0 + C - 1) // C

    zv = jnp.zeros((16,), jnp.float32)
    for s in range(SPW):
        for k in range(NREG):
            acc[s, pl.ds(16 * k, 16)] = zv

    def flush(c, regs):
        srel = c - seg_lo
        for k in range(NREG):
            acc[srel, pl.ds(16 * k, 16)] += regs[k]

    bufs = ((h_va, b_va, l_va, gm_va, go_va, sem_a),
            (h_vb, b_vb, l_vb, gm_vb, go_vb, sem_b))

    def chunk_base(kc):
        return jnp.minimum(a0 + kc * C, N - C)

    def start_chunk(kc, par):
        base = chunk_base(kc)
        hv, bv, lv, gv, _, sem = bufs[par]
        pltpu.async_copy(h_hbm.at[pl.ds(base, C)], hv, sem)
        pltpu.async_copy(batch_hbm.at[pl.ds(base, C)], bv, sem)
        pltpu.async_copy(logit_hbm.at[pl.ds(base, C)], lv, sem)
        pltpu.async_copy(gum_hbm.at[pl.ds(2 * base, 2 * C)], gv, sem)

    def wait_chunk(kc, par):
        base = chunk_base(kc)
        hv, bv, lv, gv, _, sem = bufs[par]
        pltpu.make_async_copy(h_hbm.at[pl.ds(base, C)], hv, sem).wait()
        pltpu.make_async_copy(batch_hbm.at[pl.ds(base, C)], bv, sem).wait()
        pltpu.make_async_copy(logit_hbm.at[pl.ds(base, C)], lv, sem).wait()
        pltpu.make_async_copy(gum_hbm.at[pl.ds(2 * base, 2 * C)],
                              gv, sem).wait()

    @pl.when(nch > 0)
    def _():
        start_chunk(0, 0)

    iota16 = _IOTA()
    idx_e = (2 * iota16) & 15
    idx_o = (2 * iota16 + 1) & 15
    half = iota16 < 8

    def process_chunk(kc, par, carry):
        h_vm, b_vm, l_vm, gm_vm, go_vm, _ = bufs[par]
        base = chunk_base(kc)
        skip = (a0 + kc * C) - base

        def grp_body(gi, carry2):
            cur2, regs2 = carry2
            o = gi * 16
            bvec = b_vm[pl.ds(o, 16)]
            lvec = l_vm[pl.ds(o, 16)]
            ga = gm_vm[pl.ds(2 * o, 16)]
            gb2 = gm_vm[pl.ds(2 * o + 16, 16)]
            ge = jnp.where(half, jnp.take_along_axis(ga, idx_e, axis=0),
                           jnp.take_along_axis(gb2, idx_e, axis=0))
            go_ = jnp.where(half, jnp.take_along_axis(ga, idx_o, axis=0),
                            jnp.take_along_axis(gb2, idx_o, axis=0))
            gvec = _sigmoid16(lvec + (go_ - ge))
            go_vm[pl.ds(o, 16)] = gvec
            jvec = _IOTA() + o
            valid = ((bvec >= seg_lo) & (bvec < seg_lo + SPW)
                     & (jvec >= skip))
            b0 = _ext_i(bvec, 0)
            unif = jnp.all((bvec == b0) & valid)

            def fast(args):
                cur3, regs3 = args

                def do_flush(a2):
                    c4, r4 = a2

                    @pl.when(c4 >= 0)
                    def _():
                        flush(c4, r4)

                    return (b0, _zero_regs())

                cur4, regs4 = lax.cond(
                    b0 != cur3, do_flush, lambda a2: a2, (cur3, regs3))
                regs5 = list(regs4)
                for j in range(16):
                    gb = _bcast_lane(gvec, j)
                    for k in range(D // 16):
                        hv = h_vm[o + j, pl.ds(16 * k, 16)]
                        regs5[k] = regs5[k] + gb * hv
                        regs5[8 + k] = regs5[8 + k] + hv
                regs5[16] = regs5[16] + gvec
                return (b0, tuple(regs5))

            def slow(args):
                cur3, regs3 = args

                @pl.when(cur3 >= 0)
                def _():
                    flush(cur3, regs3)

                vnum = valid.astype(jnp.int32)

                def row_body(j, _):
                    vj = _ext_i(vnum, j)

                    @pl.when(vj > 0)
                    def _():
                        bj = _ext_i(bvec, j)
                        gj = _ext_f(gvec, j)
                        srel = bj - seg_lo
                        gb = jnp.full((16,), gj, jnp.float32)
                        for k in range(D // 16):
                            hv = h_vm[o + j, pl.ds(16 * k, 16)]
                            acc[srel, pl.ds(16 * k, 16)] += gb * hv
                            acc[srel, pl.ds(D + 16 * k, 16)] += hv
                        acc[srel, pl.ds(2 * D, 16)] += jnp.where(
                            _IOTA() == 0, gb, 0.0)

                    return 0

                lax.fori_loop(0, 16, row_body, 0)
                return (jnp.int32(-1), _zero_regs())

            return lax.cond(unif, fast, slow, (cur2, regs2))

        carry = lax.fori_loop(0, GRPS, grp_body, carry)
        pltpu.sync_copy(go_vm, gate_hbm.at[pl.ds(base, C)])
        return carry

    def pair_body(kp, carry):
        def one(kc, par, carry):
            def go(carry):
                @pl.when(kc + 1 < nch)
                def _():
                    start_chunk(kc + 1, 1 - par)

                wait_chunk(kc, par)
                return process_chunk(kc, par, carry)

            return lax.cond(kc < nch, go, lambda c: c, carry)

        carry = one(2 * kp, 0, carry)
        carry = one(2 * kp + 1, 1, carry)
        return carry

    npairs = (nch + 1) // 2
    cur_f, regs_f = lax.fori_loop(
        0, npairs, pair_body, (jnp.int32(-1), _zero_regs()))

    @pl.when(cur_f >= 0)
    def _():
        flush(cur_f, regs_f)

    rvec = jnp.zeros((16,), jnp.float32)
    evec = jnp.zeros((16,), jnp.float32)
    for s in range(SPW):
        cnt = (starts[s + 1] - starts[s]).astype(jnp.float32)
        ccv = jnp.maximum(jnp.full((16,), cnt, jnp.float32), 1.0)
        iv = jnp.ones((16,), jnp.float32) / ccv
        for k in range(D // 16):
            gh = acc[s, pl.ds(16 * k, 16)]
            hh = acc[s, pl.ds(D + 16 * k, 16)]
            hbuf[s, pl.ds(16 * k, 16)] = gh * iv
            cbuf[s, pl.ds(16 * k, 16)] = (hh - gh) * iv
        sg = jnp.sum(acc[s, pl.ds(2 * D, 16)])
        sel = _IOTA() == s
        rvec = jnp.where(sel, sg + 1e-8, rvec)
        evec = jnp.where(sel, cnt - sg + 1e-8, evec)
    rbuf[pl.ds(0, 16)] = rvec
    rbuf[pl.ds(16, 16)] = evec
    pltpu.sync_copy(hbuf, hout_hbm.at[pl.ds(seg_lo, SPW)])
    pltpu.sync_copy(cbuf, cout_hbm.at[pl.ds(seg_lo, SPW)])
    pltpu.sync_copy(rbuf.at[pl.ds(0, 16)], r_hbm.at[pl.ds(seg_lo, SPW)])
    pltpu.sync_copy(rbuf.at[pl.ds(16, 16)], env_hbm.at[pl.ds(seg_lo, SPW)])


def _sc_call(h_node, logit_flat, batch_i32, gum_flat, off_flat):
    mesh = plsc.VectorSubcoreMesh(
        core_axis_name="c", subcore_axis_name="s",
        num_cores=NC, num_subcores=NS)
    f = pl.kernel(
        _sc_body,
        out_type=(
            jax.ShapeDtypeStruct((G, D), jnp.float32),
            jax.ShapeDtypeStruct((G, D), jnp.float32),
            jax.ShapeDtypeStruct((G,), jnp.float32),
            jax.ShapeDtypeStruct((G,), jnp.float32),
            jax.ShapeDtypeStruct((N,), jnp.float32),
        ),
        mesh=mesh,
        compiler_params=pltpu.CompilerParams(needs_layout_passes=False),
        scratch_types=[
            pltpu.VMEM((C, D), jnp.float32),
            pltpu.VMEM((C, D), jnp.float32),
            pltpu.VMEM((C,), jnp.int32),
            pltpu.VMEM((C,), jnp.int32),
            pltpu.VMEM((C,), jnp.float32),
            pltpu.VMEM((C,), jnp.float32),
            pltpu.VMEM((2 * C,), jnp.float32),
            pltpu.VMEM((2 * C,), jnp.float32),
            pltpu.VMEM((C,), jnp.float32),
            pltpu.VMEM((C,), jnp.float32),
            pltpu.VMEM((32,), jnp.int32),
            pltpu.VMEM((SPW, NREG * 16), jnp.float32),
            pltpu.VMEM((SPW, D), jnp.float32),
            pltpu.VMEM((SPW, D), jnp.float32),
            pltpu.VMEM((32,), jnp.float32),
            pltpu.SemaphoreType.DMA,
            pltpu.SemaphoreType.DMA,
        ],
    )
    return f(h_node, logit_flat, batch_i32, gum_flat, off_flat)


@functools.partial(jax.jit, static_argnames=())
def kernel(x_in, h_node, batch, W_gnn, b_gnn, W_gate, b_gate, g):
    batch_i32 = batch.astype(jnp.int32)
    batch3d = batch_i32.reshape(NB, 1, B)
    b_gnn2 = b_gnn.reshape(1, D)
    b_gate2 = b_gate.reshape(1, 2)

    logit, off = _tc_call(
        x_in, batch3d, W_gnn, b_gnn2, W_gate, b_gate2)

    h_out, c_out, r_flat, env_flat, gate_flat = _sc_call(
        h_node, logit.reshape(N), batch_i32, g.reshape(2 * N),
        off.reshape(OFF_PAD))
    return (h_out, c_out, r_flat.reshape(G, 1), env_flat.reshape(G, 1),
            gate_flat.reshape(N, 1))


# speedup vs baseline: 1.4006x; 1.4006x over previous
import functools

import jax
import jax.numpy as jnp
from jax import lax
from jax.experimental import pallas as pl
from jax.experimental.pallas import tpu as pltpu
from jax.experimental.pallas import tpu_sc as plsc

N = 100000
D = 128
G = 512
B = 5000
NB = N // B
OFF_PAD = 544
HI = 4
LO = 128

NC = 2
NS = 16
NW = NC * NS
SPW = G // NW
C = 384
GRPS = C // 16
NREG = 17


def _tc_body(x_ref, b_ref, wg_ref, bg_ref, wgate_ref, bgate_ref,
             logit_ref, off_ref, acc_c):
    i = pl.program_id(0)

    @pl.when(i == 0)
    def _init():
        acc_c[...] = jnp.zeros_like(acc_c)

    dvec = jnp.concatenate(
        [jnp.full((1, 1), -1.0, jnp.float32),
         jnp.full((1, 1), 1.0, jnp.float32)], axis=0)
    wd = jax.lax.dot(wgate_ref[...], dvec,
                     preferred_element_type=jnp.float32)
    bd = jax.lax.dot(bgate_ref[...], dvec,
                     preferred_element_type=jnp.float32)
    vt = jax.lax.dot_general(
        wd, wg_ref[...], (((0,), (1,)), ((), ())),
        preferred_element_type=jnp.float32)
    c = jax.lax.dot_general(
        bg_ref[...], vt, (((1,), (1,)), ((), ())),
        preferred_element_type=jnp.float32)
    logit_t = (jax.lax.dot_general(
                   vt, x_ref[...], (((1,), (1,)), ((), ())),
                   preferred_element_type=jnp.float32)
               + (c[0, 0] + bd[0, 0]))
    logit_ref[0] = logit_t

    ids = b_ref[0]
    hi_t = ids >> 7
    lo_t = ids & 127
    ihi = jax.lax.broadcasted_iota(jnp.int32, (HI, B), 0)
    ilo = jax.lax.broadcasted_iota(jnp.int32, (LO, B), 0)
    ohhi = (hi_t == ihi).astype(jnp.float32)
    ohlo = (lo_t == ilo).astype(jnp.float32)
    acc_c[...] += jax.lax.dot_general(
        ohhi, ohlo, (((1,), (1,)), ((), ())),
        preferred_element_type=jnp.float32)

    @pl.when(i == NB - 1)
    def _finalize():
        shi = jax.lax.broadcasted_iota(jnp.int32, (G, HI), 0) >> 7
        ih = jax.lax.broadcasted_iota(jnp.int32, (G, HI), 1)
        p = (shi == ih).astype(jnp.float32)
        slo = jax.lax.broadcasted_iota(jnp.int32, (G, LO), 0) & 127
        il = jax.lax.broadcasted_iota(jnp.int32, (G, LO), 1)
        rsel = (slo == il).astype(jnp.float32)
        t = jax.lax.dot(p, acc_c[...],
                        preferred_element_type=jnp.float32)
        count = jnp.sum(t * rsel, axis=1, keepdims=True)
        it = jax.lax.broadcasted_iota(jnp.int32, (G, OFF_PAD), 0)
        ic = jax.lax.broadcasted_iota(jnp.int32, (G, OFF_PAD), 1)
        m = (it < ic).astype(jnp.float32)
        off = jax.lax.dot_general(
            count, m, (((0,), (0,)), ((), ())),
            preferred_element_type=jnp.float32)
        off_ref[...] = off.astype(jnp.int32)


def _tc_call(x_in, batch_3d, W_gnn, b_gnn2, W_gate, b_gate2):
    out_shapes = (
        jax.ShapeDtypeStruct((NB, 1, B), jnp.float32),
        jax.ShapeDtypeStruct((1, OFF_PAD), jnp.int32),
    )
    in_specs = [
        pl.BlockSpec((B, D), lambda i: (i, 0)),
        pl.BlockSpec((1, 1, B), lambda i: (i, 0, 0)),
        pl.BlockSpec((D, D), lambda i: (0, 0)),
        pl.BlockSpec((1, D), lambda i: (0, 0)),
        pl.BlockSpec((D, 2), lambda i: (0, 0)),
        pl.BlockSpec((1, 2), lambda i: (0, 0)),
    ]
    out_specs = (
        pl.BlockSpec((1, 1, B), lambda i: (i, 0, 0)),
        pl.BlockSpec((1, OFF_PAD), lambda i: (0, 0)),
    )
    return pl.pallas_call(
        _tc_body,
        grid=(NB,),
        in_specs=in_specs,
        out_specs=out_specs,
        out_shape=out_shapes,
        scratch_shapes=[pltpu.VMEM((HI, LO), jnp.float32)],
    )(x_in, batch_3d, W_gnn, b_gnn2, W_gate, b_gate2)


_IOTA = lambda: lax.iota(jnp.int32, 16)


def _ext_i(vec, j):
    return jnp.sum(jnp.where(_IOTA() == j, vec, 0))


def _ext_f(vec, j):
    return jnp.sum(jnp.where(_IOTA() == j, vec, 0.0))


def _bcast_lane(vec, j):
    idx = jnp.full((16,), j, jnp.int32)
    return jnp.take_along_axis(vec, idx, axis=0)


def _zero_regs():
    return tuple(jnp.zeros((16,), jnp.float32) for _ in range(NREG))


def _sigmoid16(x):
    return jnp.ones((16,), jnp.float32) / (1.0 + jnp.exp(-x))


def _sc_body(h_hbm, logit_hbm, batch_hbm, g0_hbm, g1_hbm, off_hbm,
             hout_hbm, cout_hbm, r_hbm, env_hbm, gate_hbm,
             h_va, h_vb, b_va, b_vb, l_va, l_vb, gm_va, gm_vb, go_va, go_vb,
             off_vmem, acc, hbuf, cbuf, rbuf,
             sem_a, sem_b):
    cid = lax.axis_index("c")
    sid = lax.axis_index("s")
    w = sid * NC + cid
    seg_lo = w * SPW

    pltpu.sync_copy(off_hbm.at[pl.ds(seg_lo, 32)], off_vmem)
    ovec0 = off_vmem[pl.ds(0, 16)]
    ovec1 = off_vmem[pl.ds(16, 16)]
    starts = [_ext_i(ovec0, j) for j in range(16)] + [_ext_i(ovec1, 0)]
    row_lo = starts[0]
    row_hi = starts[SPW]
    a0 = (row_lo // 16) * 16
    z0 = ((row_hi + 15) // 16) * 16
    nch = (z0 - a0 + C - 1) // C

    zv = jnp.zeros((16,), jnp.float32)
    for s in range(SPW):
        for k in range(NREG):
            acc[s, pl.ds(16 * k, 16)] = zv

    def flush(c, regs):
        srel = c - seg_lo
        for k in range(NREG):
            acc[srel, pl.ds(16 * k, 16)] += regs[k]

    bufs = ((h_va, b_va, l_va, gm_va, go_va, sem_a),
            (h_vb, b_vb, l_vb, gm_vb, go_vb, sem_b))

    def chunk_base(kc):
        return jnp.minimum(a0 + kc * C, N - C)

    def start_chunk(kc, par):
        base = chunk_base(kc)
        hv, bv, lv, gv, _, sem = bufs[par]
        pltpu.async_copy(h_hbm.at[pl.ds(base, C)], hv, sem)
        pltpu.async_copy(batch_hbm.at[pl.ds(base, C)], bv, sem)
        pltpu.async_copy(logit_hbm.at[pl.ds(base, C)], lv, sem)
        pltpu.async_copy(g0_hbm.at[pl.ds(base, C)], gv.at[pl.ds(0, C)], sem)
        pltpu.async_copy(g1_hbm.at[pl.ds(base, C)], gv.at[pl.ds(C, C)], sem)

    def wait_chunk(kc, par):
        base = chunk_base(kc)
        hv, bv, lv, gv, _, sem = bufs[par]
        pltpu.make_async_copy(h_hbm.at[pl.ds(base, C)], hv, sem).wait()
        pltpu.make_async_copy(batch_hbm.at[pl.ds(base, C)], bv, sem).wait()
        pltpu.make_async_copy(logit_hbm.at[pl.ds(base, C)], lv, sem).wait()
        pltpu.make_async_copy(g0_hbm.at[pl.ds(base, C)],
                              gv.at[pl.ds(0, C)], sem).wait()
        pltpu.make_async_copy(g1_hbm.at[pl.ds(base, C)],
                              gv.at[pl.ds(C, C)], sem).wait()

    @pl.when(nch > 0)
    def _():
        start_chunk(0, 0)

    def process_chunk(kc, par, carry):
        h_vm, b_vm, l_vm, gm_vm, go_vm, _ = bufs[par]
        base = chunk_base(kc)
        skip = (a0 + kc * C) - base

        def grp_body(gi, carry2):
            cur2, regs2 = carry2
            o = gi * 16
            bvec = b_vm[pl.ds(o, 16)]
            lvec = l_vm[pl.ds(o, 16)]
            g0v = gm_vm[pl.ds(o, 16)]
            g1v = gm_vm[pl.ds(C + o, 16)]
            gvec = _sigmoid16(lvec + (g1v - g0v))
            go_vm[pl.ds(o, 16)] = gvec
            jvec = _IOTA() + o
            valid = ((bvec >= seg_lo) & (bvec < seg_lo + SPW)
                     & (jvec >= skip))
            b0 = _ext_i(bvec, 0)
            unif = jnp.all((bvec == b0) & valid)

            def fast(args):
                cur3, regs3 = args

                def do_flush(a2):
                    c4, r4 = a2

                    @pl.when(c4 >= 0)
                    def _():
                        flush(c4, r4)

                    return (b0, _zero_regs())

                cur4, regs4 = lax.cond(
                    b0 != cur3, do_flush, lambda a2: a2, (cur3, regs3))
                regs5 = list(regs4)
                for j in range(16):
                    gb = _bcast_lane(gvec, j)
                    for k in range(D // 16):
                        hv = h_vm[o + j, pl.ds(16 * k, 16)]
                        regs5[k] = regs5[k] + gb * hv
                        regs5[8 + k] = regs5[8 + k] + hv
                regs5[16] = regs5[16] + gvec
                return (b0, tuple(regs5))

            def slow(args):
                cur3, regs3 = args

                @pl.when(cur3 >= 0)
                def _():
                    flush(cur3, regs3)

                vnum = valid.astype(jnp.int32)

                def row_body(j, _):
                    vj = _ext_i(vnum, j)

                    @pl.when(vj > 0)
                    def _():
                        bj = _ext_i(bvec, j)
                        gj = _ext_f(gvec, j)
                        srel = bj - seg_lo
                        gb = jnp.full((16,), gj, jnp.float32)
                        for k in range(D // 16):
                            hv = h_vm[o + j, pl.ds(16 * k, 16)]
                            acc[srel, pl.ds(16 * k, 16)] += gb * hv
                            acc[srel, pl.ds(D + 16 * k, 16)] += hv
                        acc[srel, pl.ds(2 * D, 16)] += jnp.where(
                            _IOTA() == 0, gb, 0.0)

                    return 0

                lax.fori_loop(0, 16, row_body, 0)
                return (jnp.int32(-1), _zero_regs())

            return lax.cond(unif, fast, slow, (cur2, regs2))

        carry = lax.fori_loop(0, GRPS, grp_body, carry)
        pltpu.sync_copy(go_vm, gate_hbm.at[pl.ds(base, C)])
        return carry

    def pair_body(kp, carry):
        def one(kc, par, carry):
            def go(carry):
                @pl.when(kc + 1 < nch)
                def _():
                    start_chunk(kc + 1, 1 - par)

                wait_chunk(kc, par)
                return process_chunk(kc, par, carry)

            return lax.cond(kc < nch, go, lambda c: c, carry)

        carry = one(2 * kp, 0, carry)
        carry = one(2 * kp + 1, 1, carry)
        return carry

    npairs = (nch + 1) // 2
    cur_f, regs_f = lax.fori_loop(
        0, npairs, pair_body, (jnp.int32(-1), _zero_regs()))

    @pl.when(cur_f >= 0)
    def _():
        flush(cur_f, regs_f)

    rvec = jnp.zeros((16,), jnp.float32)
    evec = jnp.zeros((16,), jnp.float32)
    for s in range(SPW):
        cnt = (starts[s + 1] - starts[s]).astype(jnp.float32)
        ccv = jnp.maximum(jnp.full((16,), cnt, jnp.float32), 1.0)
        iv = jnp.ones((16,), jnp.float32) / ccv
        for k in range(D // 16):
            gh = acc[s, pl.ds(16 * k, 16)]
            hh = acc[s, pl.ds(D + 16 * k, 16)]
            hbuf[s, pl.ds(16 * k, 16)] = gh * iv
            cbuf[s, pl.ds(16 * k, 16)] = (hh - gh) * iv
        sg = jnp.sum(acc[s, pl.ds(2 * D, 16)])
        sel = _IOTA() == s
        rvec = jnp.where(sel, sg + 1e-8, rvec)
        evec = jnp.where(sel, cnt - sg + 1e-8, evec)
    rbuf[pl.ds(0, 16)] = rvec
    rbuf[pl.ds(16, 16)] = evec
    pltpu.sync_copy(hbuf, hout_hbm.at[pl.ds(seg_lo, SPW)])
    pltpu.sync_copy(cbuf, cout_hbm.at[pl.ds(seg_lo, SPW)])
    pltpu.sync_copy(rbuf.at[pl.ds(0, 16)], r_hbm.at[pl.ds(seg_lo, SPW)])
    pltpu.sync_copy(rbuf.at[pl.ds(16, 16)], env_hbm.at[pl.ds(seg_lo, SPW)])


def _sc_call(h_node, logit_flat, batch_i32, g0, g1, off_flat):
    mesh = plsc.VectorSubcoreMesh(
        core_axis_name="c", subcore_axis_name="s",
        num_cores=NC, num_subcores=NS)
    f = pl.kernel(
        _sc_body,
        out_type=(
            jax.ShapeDtypeStruct((G, D), jnp.float32),
            jax.ShapeDtypeStruct((G, D), jnp.float32),
            jax.ShapeDtypeStruct((G,), jnp.float32),
            jax.ShapeDtypeStruct((G,), jnp.float32),
            jax.ShapeDtypeStruct((N,), jnp.float32),
        ),
        mesh=mesh,
        compiler_params=pltpu.CompilerParams(needs_layout_passes=False),
        scratch_types=[
            pltpu.VMEM((C, D), jnp.float32),
            pltpu.VMEM((C, D), jnp.float32),
            pltpu.VMEM((C,), jnp.int32),
            pltpu.VMEM((C,), jnp.int32),
            pltpu.VMEM((C,), jnp.float32),
            pltpu.VMEM((C,), jnp.float32),
            pltpu.VMEM((2 * C,), jnp.float32),
            pltpu.VMEM((2 * C,), jnp.float32),
            pltpu.VMEM((C,), jnp.float32),
            pltpu.VMEM((C,), jnp.float32),
            pltpu.VMEM((32,), jnp.int32),
            pltpu.VMEM((SPW, NREG * 16), jnp.float32),
            pltpu.VMEM((SPW, D), jnp.float32),
            pltpu.VMEM((SPW, D), jnp.float32),
            pltpu.VMEM((32,), jnp.float32),
            pltpu.SemaphoreType.DMA,
            pltpu.SemaphoreType.DMA,
        ],
    )
    return f(h_node, logit_flat, batch_i32, g0, g1, off_flat)


@functools.partial(jax.jit, static_argnames=())
def kernel(x_in, h_node, batch, W_gnn, b_gnn, W_gate, b_gate, g):
    batch_i32 = batch.astype(jnp.int32)
    batch3d = batch_i32.reshape(NB, 1, B)
    b_gnn2 = b_gnn.reshape(1, D)
    b_gate2 = b_gate.reshape(1, 2)

    logit, off = _tc_call(
        x_in, batch3d, W_gnn, b_gnn2, W_gate, b_gate2)

    h_out, c_out, r_flat, env_flat, gate_flat = _sc_call(
        h_node, logit.reshape(N), batch_i32, g[:, 0], g[:, 1],
        off.reshape(OFF_PAD))
    return (h_out, c_out, r_flat.reshape(G, 1), env_flat.reshape(G, 1),
            gate_flat.reshape(N, 1))
